# baseline (speedup 1.0000x reference)
import functools

import jax
import jax.numpy as jnp
from jax import lax
from jax.experimental import pallas as pl
from jax.experimental.pallas import tpu as pltpu
from jax.experimental.pallas import tpu_sc as plsc

N = 10000
E = 320000
D = 128
DE = 16
CH = 80
NW = 32
EPW = E // NW
ITERS = EPW // CH
NP = 10240
RPW = NP // 16

_f32 = jnp.float32

@functools.lru_cache(maxsize=None)
def _sc_mesh():
    return plsc.VectorSubcoreMesh(core_axis_name="c", subcore_axis_name="s")


def _agg_nocnt_body(p_hbm, src_hbm, dst_hbm, z128_hbm, agg_out,
                    sidx, didx0, didx1, rows0, rows1, acc_sh,
                    sr0, sr1, sd0, sd1):
    c = lax.axis_index("c")
    s = lax.axis_index("s")
    wid = s * 2 + c
    e0 = wid * EPW
    r0 = s * RPW
    pltpu.sync_copy(z128_hbm.at[pl.ds(r0, RPW)], acc_sh.at[pl.ds(r0, RPW)])
    pltpu.sync_copy(src_hbm.at[pl.ds(e0, EPW)], sidx)
    plsc.subcore_barrier()

    def issue(i, rows, semr, didxb, semd):
        pltpu.async_copy(dst_hbm.at[pl.ds(e0 + i * CH, CH)], didxb, semd)
        pltpu.async_copy(p_hbm.at[sidx.at[pl.ds(i * CH, CH)]], rows, semr)

    def drain(i, rows, semr, didxb, semd):
        pltpu.make_async_copy(dst_hbm.at[pl.ds(e0 + i * CH, CH)], didxb, semd).wait()
        pltpu.make_async_copy(p_hbm.at[sidx.at[pl.ds(i * CH, CH)]], rows, semr).wait()
        pltpu.sync_copy(rows, acc_sh.at[didxb], add=True)

    issue(0, rows0, sr0, didx0, sd0)

    def step(g, carry):
        i = g * 2
        issue(i + 1, rows1, sr1, didx1, sd1)
        drain(i, rows0, sr0, didx0, sd0)
        issue(i + 2, rows0, sr0, didx0, sd0)
        drain(i + 1, rows1, sr1, didx1, sd1)
        return carry

    lax.fori_loop(0, (ITERS - 1) // 2, step, 0)
    drain(ITERS - 1, rows0, sr0, didx0, sd0)
    plsc.subcore_barrier()
    pltpu.sync_copy(acc_sh.at[pl.ds(r0, RPW)], agg_out.at[c, pl.ds(r0, RPW)])


@functools.lru_cache(maxsize=None)
def _agg_kernel():
    return pl.kernel(
        _agg_nocnt_body,
        out_type=jax.ShapeDtypeStruct((2, NP, D), _f32),
        mesh=_sc_mesh(),
        scratch_types=[pltpu.VMEM((EPW,), jnp.int32),
                       pltpu.VMEM((CH,), jnp.int32),
                       pltpu.VMEM((CH,), jnp.int32),
                       pltpu.VMEM((CH, D), _f32),
                       pltpu.VMEM((CH, D), _f32),
                       pltpu.VMEM_SHARED((NP, D), _f32),
                       pltpu.SemaphoreType.DMA,
                       pltpu.SemaphoreType.DMA,
                       pltpu.SemaphoreType.DMA,
                       pltpu.SemaphoreType.DMA])


def _cnt_body(dst_hbm, ones_hbm, z128_hbm, cnt_out,
              didx0, didx1, ones_v, acc_sh, sd0, sd1):
    c = lax.axis_index("c")
    s = lax.axis_index("s")
    wid = s * 2 + c
    e0 = wid * EPW
    r0 = s * RPW
    pltpu.sync_copy(z128_hbm.at[pl.ds(r0, RPW)], acc_sh.at[pl.ds(r0, RPW)])
    pltpu.sync_copy(ones_hbm, ones_v)
    plsc.subcore_barrier()

    def issue(i, didxb, semd):
        pltpu.async_copy(dst_hbm.at[pl.ds(e0 + i * CH, CH)], didxb, semd)

    def drain(i, didxb, semd):
        pltpu.make_async_copy(dst_hbm.at[pl.ds(e0 + i * CH, CH)], didxb, semd).wait()
        pltpu.sync_copy(ones_v, acc_sh.at[didxb], add=True)

    issue(0, didx0, sd0)

    def step(g, carry):
        i = g * 2
        issue(i + 1, didx1, sd1)
        drain(i, didx0, sd0)
        issue(i + 2, didx0, sd0)
        drain(i + 1, didx1, sd1)
        return carry

    lax.fori_loop(0, (ITERS - 1) // 2, step, 0)
    drain(ITERS - 1, didx0, sd0)
    plsc.subcore_barrier()
    pltpu.sync_copy(acc_sh.at[pl.ds(r0, RPW)], cnt_out.at[c, pl.ds(r0, RPW)])


@functools.lru_cache(maxsize=None)
def _cnt_kernel():
    return pl.kernel(
        _cnt_body,
        out_type=jax.ShapeDtypeStruct((2, NP, D), _f32),
        mesh=_sc_mesh(),
        scratch_types=[pltpu.VMEM((CH,), jnp.int32),
                       pltpu.VMEM((CH,), jnp.int32),
                       pltpu.VMEM((CH, D), _f32),
                       pltpu.VMEM_SHARED((NP, D), _f32),
                       pltpu.SemaphoreType.DMA,
                       pltpu.SemaphoreType.DMA])


def _make_edge_gather(ne, ch):
    epw = ne // NW
    iters = epw // ch

    def body(a_hbm, b_hbm, src_hbm, dst_hbm, ga_out, gb_out,
             sidx, didx, ra0, ra1, rb0, rb1, sa0, sa1, sb0, sb1):
        c = lax.axis_index("c")
        s = lax.axis_index("s")
        wid = s * 2 + c
        e0 = wid * epw
        pltpu.sync_copy(src_hbm.at[pl.ds(e0, epw)], sidx)
        pltpu.sync_copy(dst_hbm.at[pl.ds(e0, epw)], didx)

        def issue(i, bufa, sema, bufb, semb):
            pltpu.async_copy(a_hbm.at[sidx.at[pl.ds(i * ch, ch)]], bufa, sema)
            pltpu.async_copy(b_hbm.at[didx.at[pl.ds(i * ch, ch)]], bufb, semb)

        def drain(i, bufa, sema, bufb, semb):
            base = e0 + i * ch
            pltpu.make_async_copy(a_hbm.at[sidx.at[pl.ds(i * ch, ch)]], bufa, sema).wait()
            pltpu.sync_copy(bufa, ga_out.at[pl.ds(base, ch)])
            pltpu.make_async_copy(b_hbm.at[didx.at[pl.ds(i * ch, ch)]], bufb, semb).wait()
            pltpu.sync_copy(bufb, gb_out.at[pl.ds(base, ch)])

        issue(0, ra0, sa0, rb0, sb0)

        def step(g, carry):
            i = g * 2
            issue(i + 1, ra1, sa1, rb1, sb1)
            drain(i, ra0, sa0, rb0, sb0)
            issue(i + 2, ra0, sa0, rb0, sb0)
            drain(i + 1, ra1, sa1, rb1, sb1)
            return carry

        lax.fori_loop(0, (iters - 1) // 2, step, 0)
        drain(iters - 1, ra0, sa0, rb0, sb0)

    return pl.kernel(
        body,
        out_type=[jax.ShapeDtypeStruct((ne, D), _f32),
                  jax.ShapeDtypeStruct((ne, D), _f32)],
        mesh=_sc_mesh(),
        scratch_types=[pltpu.VMEM((epw,), jnp.int32),
                       pltpu.VMEM((epw,), jnp.int32),
                       pltpu.VMEM((ch, D), _f32),
                       pltpu.VMEM((ch, D), _f32),
                       pltpu.VMEM((ch, D), _f32),
                       pltpu.VMEM((ch, D), _f32),
                       pltpu.SemaphoreType.DMA,
                       pltpu.SemaphoreType.DMA,
                       pltpu.SemaphoreType.DMA,
                       pltpu.SemaphoreType.DMA])


@functools.lru_cache(maxsize=None)
def _edge_gather_kernel(ne, ch):
    return _make_edge_gather(ne, ch)


_NB = 1000
_EB = 4000


def _mm_body(x_ref, w_ref, o_ref):
    o_ref[...] = jnp.dot(x_ref[...], w_ref[...],
                         preferred_element_type=_f32)


def _mm(x, w):
    return pl.pallas_call(
        _mm_body,
        grid=(N // _NB,),
        in_specs=[pl.BlockSpec((_NB, D), lambda i: (i, 0)),
                  pl.BlockSpec((D, D), lambda i: (0, 0))],
        out_specs=pl.BlockSpec((_NB, D), lambda i: (i, 0)),
        out_shape=jax.ShapeDtypeStruct((N, D), _f32),
    )(x, w)


def _finish_body(p0_ref, p1_ref, c0_ref, c1_ref, hprev_ref, wr_ref, bl_ref,
                 wa_ref, wb_ref, a_ref, b_ref):
    s = p0_ref[...] + p1_ref[...]
    cnt = c0_ref[...][:, 0:1] + c1_ref[...][:, 0:1]
    mean = s / jnp.maximum(cnt, 1.0)
    h = mean + bl_ref[...] + jnp.dot(hprev_ref[...], wr_ref[...],
                                     preferred_element_type=_f32)
    h = jnp.maximum(h, 0.0)
    a_ref[...] = jnp.dot(h, wa_ref[...],
                         preferred_element_type=_f32).astype(a_ref.dtype)
    b_ref[...] = jnp.dot(h, wb_ref[...],
                         preferred_element_type=_f32).astype(b_ref.dtype)


def _finish_and_project(p0, p1, c0, c1, hprev, wr, bl, wa, wb, want_h):
    body = _finish_body if not want_h else _finish_h_body
    return pl.pallas_call(
        body,
        grid=(N // _NB,),
        in_specs=[pl.BlockSpec((_NB, D), lambda i: (i, 0)),
                  pl.BlockSpec((_NB, D), lambda i: (i, 0)),
                  pl.BlockSpec((_NB, D), lambda i: (i, 0)),
                  pl.BlockSpec((_NB, D), lambda i: (i, 0)),
                  pl.BlockSpec((_NB, D), lambda i: (i, 0)),
                  pl.BlockSpec((D, D), lambda i: (0, 0)),
                  pl.BlockSpec((1, D), lambda i: (0, 0)),
                  pl.BlockSpec((D, D), lambda i: (0, 0)),
                  pl.BlockSpec((D, D), lambda i: (0, 0))],
        out_specs=[pl.BlockSpec((_NB, D), lambda i: (i, 0)),
                   pl.BlockSpec((_NB, D), lambda i: (i, 0))],
        out_shape=[jax.ShapeDtypeStruct((N, D), _f32),
                   jax.ShapeDtypeStruct((N, D), _f32)],
    )(p0, p1, c0, c1, hprev, wr, bl, wa, wb)


def _finish_h_body(p0_ref, p1_ref, c0_ref, c1_ref, hprev_ref, wr_ref, bl_ref,
                   wa_ref, wb_ref, h_ref, b_ref):
    s = p0_ref[...] + p1_ref[...]
    cnt = c0_ref[...][:, 0:1] + c1_ref[...][:, 0:1]
    mean = s / jnp.maximum(cnt, 1.0)
    h = mean + bl_ref[...] + jnp.dot(hprev_ref[...], wr_ref[...],
                                     preferred_element_type=_f32)
    h = jnp.maximum(h, 0.0)
    h_ref[...] = h
    b_ref[...] = jnp.dot(h, wb_ref[...],
                         preferred_element_type=_f32).astype(b_ref.dtype)


def _head_body(ga_ref, gb_ref, ea_ref, we_ref, b1_ref, w2_ref, b2_ref, o_ref):
    pre = (ga_ref[...].astype(_f32) + gb_ref[...].astype(_f32) + b1_ref[...]
           + jnp.dot(ea_ref[...], we_ref[...], preferred_element_type=_f32))
    hid = jnp.maximum(pre, 0.0)
    o_ref[...] = jnp.sum(hid * w2_ref[...], axis=1, keepdims=True) + b2_ref[...]


def _head(ga, gb, ea, we, b1, w2row, b2):
    ne = ga.shape[0]
    return pl.pallas_call(
        _head_body,
        grid=(ne // _EB,),
        in_specs=[pl.BlockSpec((_EB, D), lambda i: (i, 0)),
                  pl.BlockSpec((_EB, D), lambda i: (i, 0)),
                  pl.BlockSpec((_EB, DE), lambda i: (i, 0)),
                  pl.BlockSpec((DE, D), lambda i: (0, 0)),
                  pl.BlockSpec((1, D), lambda i: (0, 0)),
                  pl.BlockSpec((1, D), lambda i: (0, 0)),
                  pl.BlockSpec((1, 1), lambda i: (0, 0))],
        out_specs=pl.BlockSpec((_EB, 1), lambda i: (i, 0)),
        out_shape=jax.ShapeDtypeStruct((ne, 1), _f32),
    )(ga, gb, ea, we, b1, w2row, b2)


def kernel(x, edge_index, edge_attr, Wl0, bl0, Wr0, Wl1, bl1, Wr1,
           W1, b1, W2, b2):
    z128 = jnp.zeros((NP, D), _f32)
    ones = jnp.ones((CH, D), _f32)
    src_ids = edge_index[0]
    dst_ids = edge_index[1]

    p0 = _mm(x, Wl0)
    P = _agg_kernel()(p0, src_ids, dst_ids, z128)
    C = _cnt_kernel()(dst_ids, ones, z128)
    h1, p1 = _finish_and_project(P[0], P[1], C[0], C[1], x, Wr0,
                                 bl0.reshape(1, D), Wl1, Wl1, want_h=True)
    P1 = _agg_kernel()(p1, src_ids, dst_ids, z128)
    A, B = _finish_and_project(P1[0], P1[1], C[0], C[1], h1, Wr1,
                               bl1.reshape(1, D), W1[:D], W1[D:2 * D],
                               want_h=False)
    eh = E // 2
    outs = []
    for lo in (0, eh):
        ga, gb = _edge_gather_kernel(eh, 40)(
            A, B, lax.dynamic_slice_in_dim(src_ids, lo, eh),
            lax.dynamic_slice_in_dim(dst_ids, lo, eh))
        outs.append(_head(ga, gb,
                          lax.dynamic_slice_in_dim(edge_attr, lo, eh),
                          W1[2 * D:], b1.reshape(1, D),
                          W2.reshape(1, D), b2.reshape(1, 1)))
    out = jnp.concatenate(outs, axis=0)
    return out


# speedup vs baseline: 1.0694x; 1.0694x over previous
import functools

import jax
import jax.numpy as jnp
from jax import lax
from jax.experimental import pallas as pl
from jax.experimental.pallas import tpu as pltpu
from jax.experimental.pallas import tpu_sc as plsc

N = 10000
E = 320000
D = 128
DE = 16
CH = 80
NW = 32
EPW = E // NW
ITERS = EPW // CH
NP = 10240
RPW = NP // 16

_f32 = jnp.float32

@functools.lru_cache(maxsize=None)
def _sc_mesh():
    return plsc.VectorSubcoreMesh(core_axis_name="c", subcore_axis_name="s")


def _run_pipeline(iters, issue, wait_in, put_out, wait_out):
    issue(0, 0)
    issue(1, 1)
    for j in (0, 1):
        issue(j + 2, (j + 2) % 4)
        wait_in(j, j % 4)
        put_out(j, j % 4)
    for j in (2, 3):
        wait_out(j - 2, (j - 2) % 4)
        issue(j + 2, (j + 2) % 4)
        wait_in(j, j % 4)
        put_out(j, j % 4)
    lo = 4
    hi = ((iters - 3) // 4) * 4

    def body(g, carry):
        for k in range(4):
            j = lo + g * 4 + k
            wait_out(j - 2, (k + 2) % 4)
            issue(j + 2, (k + 2) % 4)
            wait_in(j, k)
            put_out(j, k)
        return carry

    lax.fori_loop(0, (hi - lo) // 4, body, 0)
    for j in range(hi, iters):
        wait_out(j - 2, (j - 2) % 4)
        if j + 2 < iters:
            issue(j + 2, (j + 2) % 4)
        wait_in(j, j % 4)
        put_out(j, j % 4)
    wait_out(iters - 2, (iters - 2) % 4)
    wait_out(iters - 1, (iters - 1) % 4)


def _agg_nocnt_body(p_hbm, src_hbm, dst_hbm, z128_hbm, agg_out,
                    sidx, didx, rows, acc_sh, semi, semd, semr, sems):
    c = lax.axis_index("c")
    s = lax.axis_index("s")
    wid = s * 2 + c
    e0 = wid * EPW
    r0 = s * RPW
    pltpu.sync_copy(z128_hbm.at[pl.ds(r0, RPW)], acc_sh.at[pl.ds(r0, RPW)])
    plsc.subcore_barrier()

    def issue(i, b):
        pltpu.async_copy(src_hbm.at[pl.ds(e0 + i * CH, CH)], sidx[b], semi[b])
        pltpu.async_copy(dst_hbm.at[pl.ds(e0 + i * CH, CH)], didx[b], semd[b])
        pltpu.make_async_copy(src_hbm.at[pl.ds(e0 + i * CH, CH)], sidx[b], semi[b]).wait()
        pltpu.async_copy(p_hbm.at[sidx[b]], rows[b], semr[b])

    def wait_in(i, b):
        pltpu.make_async_copy(dst_hbm.at[pl.ds(e0 + i * CH, CH)], didx[b], semd[b]).wait()
        pltpu.make_async_copy(p_hbm.at[sidx[b]], rows[b], semr[b]).wait()

    def put_out(i, b):
        pltpu.async_copy(rows[b], acc_sh.at[didx[b]], sems[b], add=True)

    def wait_out(i, b):
        pltpu.make_async_copy(rows[b], acc_sh.at[didx[b]], sems[b]).wait()

    _run_pipeline(ITERS, issue, wait_in, put_out, wait_out)
    plsc.subcore_barrier()
    pltpu.sync_copy(acc_sh.at[pl.ds(r0, RPW)], agg_out.at[c, pl.ds(r0, RPW)])


@functools.lru_cache(maxsize=None)
def _agg_kernel():
    return pl.kernel(
        _agg_nocnt_body,
        out_type=jax.ShapeDtypeStruct((2, NP, D), _f32),
        mesh=_sc_mesh(),
        scratch_types=[[pltpu.VMEM((CH,), jnp.int32)] * 4,
                       [pltpu.VMEM((CH,), jnp.int32)] * 4,
                       [pltpu.VMEM((CH, D), _f32)] * 4,
                       pltpu.VMEM_SHARED((NP, D), _f32),
                       [pltpu.SemaphoreType.DMA] * 4,
                       [pltpu.SemaphoreType.DMA] * 4,
                       [pltpu.SemaphoreType.DMA] * 4,
                       [pltpu.SemaphoreType.DMA] * 4])


def _cnt_body(dst_hbm, ones_hbm, z128_hbm, cnt_out,
              didx0, didx1, ones_v, acc_sh, sd0, sd1):
    c = lax.axis_index("c")
    s = lax.axis_index("s")
    wid = s * 2 + c
    e0 = wid * EPW
    r0 = s * RPW
    pltpu.sync_copy(z128_hbm.at[pl.ds(r0, RPW)], acc_sh.at[pl.ds(r0, RPW)])
    pltpu.sync_copy(ones_hbm, ones_v)
    plsc.subcore_barrier()

    def issue(i, didxb, semd):
        pltpu.async_copy(dst_hbm.at[pl.ds(e0 + i * CH, CH)], didxb, semd)

    def drain(i, didxb, semd):
        pltpu.make_async_copy(dst_hbm.at[pl.ds(e0 + i * CH, CH)], didxb, semd).wait()
        pltpu.sync_copy(ones_v, acc_sh.at[didxb], add=True)

    issue(0, didx0, sd0)

    def step(g, carry):
        i = g * 2
        issue(i + 1, didx1, sd1)
        drain(i, didx0, sd0)
        issue(i + 2, didx0, sd0)
        drain(i + 1, didx1, sd1)
        return carry

    lax.fori_loop(0, (ITERS - 1) // 2, step, 0)
    drain(ITERS - 1, didx0, sd0)
    plsc.subcore_barrier()
    pltpu.sync_copy(acc_sh.at[pl.ds(r0, RPW)], cnt_out.at[c, pl.ds(r0, RPW)])


@functools.lru_cache(maxsize=None)
def _cnt_kernel():
    return pl.kernel(
        _cnt_body,
        out_type=jax.ShapeDtypeStruct((2, NP, D), _f32),
        mesh=_sc_mesh(),
        scratch_types=[pltpu.VMEM((CH,), jnp.int32),
                       pltpu.VMEM((CH,), jnp.int32),
                       pltpu.VMEM((CH, D), _f32),
                       pltpu.VMEM_SHARED((NP, D), _f32),
                       pltpu.SemaphoreType.DMA,
                       pltpu.SemaphoreType.DMA])


def _edge_gather_body(a_hbm, b_hbm, src_hbm, dst_hbm, ga_out, gb_out,
                      sidx, didx, bufa, bufb, sga, sgb, swa, swb):
    c = lax.axis_index("c")
    s = lax.axis_index("s")
    wid = s * 2 + c
    e0 = wid * EPW
    pltpu.sync_copy(src_hbm.at[pl.ds(e0, EPW)], sidx)
    pltpu.sync_copy(dst_hbm.at[pl.ds(e0, EPW)], didx)

    def issue(i, b):
        pltpu.async_copy(a_hbm.at[sidx.at[pl.ds(i * CH, CH)]], bufa[b], sga[b])
        pltpu.async_copy(b_hbm.at[didx.at[pl.ds(i * CH, CH)]], bufb[b], sgb[b])

    def wait_in(i, b):
        pltpu.make_async_copy(a_hbm.at[sidx.at[pl.ds(i * CH, CH)]], bufa[b], sga[b]).wait()
        pltpu.make_async_copy(b_hbm.at[didx.at[pl.ds(i * CH, CH)]], bufb[b], sgb[b]).wait()

    def put_out(i, b):
        pltpu.async_copy(bufa[b], ga_out.at[pl.ds(e0 + i * CH, CH)], swa[b])
        pltpu.async_copy(bufb[b], gb_out.at[pl.ds(e0 + i * CH, CH)], swb[b])

    def wait_out(i, b):
        pltpu.make_async_copy(bufa[b], ga_out.at[pl.ds(e0 + i * CH, CH)], swa[b]).wait()
        pltpu.make_async_copy(bufb[b], gb_out.at[pl.ds(e0 + i * CH, CH)], swb[b]).wait()

    _run_pipeline(ITERS, issue, wait_in, put_out, wait_out)


@functools.lru_cache(maxsize=None)
def _edge_gather_kernel():
    return pl.kernel(
        _edge_gather_body,
        out_type=[jax.ShapeDtypeStruct((E, D), _f32),
                  jax.ShapeDtypeStruct((E, D), _f32)],
        mesh=_sc_mesh(),
        scratch_types=[pltpu.VMEM((EPW,), jnp.int32),
                       pltpu.VMEM((EPW,), jnp.int32),
                       [pltpu.VMEM((CH, D), _f32)] * 4,
                       [pltpu.VMEM((CH, D), _f32)] * 4,
                       [pltpu.SemaphoreType.DMA] * 4,
                       [pltpu.SemaphoreType.DMA] * 4,
                       [pltpu.SemaphoreType.DMA] * 4,
                       [pltpu.SemaphoreType.DMA] * 4])


_NB = 1000
_EB = 4000


def _mm_body(x_ref, w_ref, o_ref):
    o_ref[...] = jnp.dot(x_ref[...], w_ref[...],
                         preferred_element_type=_f32)


def _mm(x, w):
    return pl.pallas_call(
        _mm_body,
        grid=(N // _NB,),
        in_specs=[pl.BlockSpec((_NB, D), lambda i: (i, 0)),
                  pl.BlockSpec((D, D), lambda i: (0, 0))],
        out_specs=pl.BlockSpec((_NB, D), lambda i: (i, 0)),
        out_shape=jax.ShapeDtypeStruct((N, D), _f32),
    )(x, w)


def _finish_body(p0_ref, p1_ref, c0_ref, c1_ref, hprev_ref, wr_ref, bl_ref,
                 wa_ref, wb_ref, a_ref, b_ref):
    s = p0_ref[...] + p1_ref[...]
    cnt = c0_ref[...][:, 0:1] + c1_ref[...][:, 0:1]
    mean = s / jnp.maximum(cnt, 1.0)
    h = mean + bl_ref[...] + jnp.dot(hprev_ref[...], wr_ref[...],
                                     preferred_element_type=_f32)
    h = jnp.maximum(h, 0.0)
    a_ref[...] = jnp.dot(h, wa_ref[...],
                         preferred_element_type=_f32).astype(a_ref.dtype)
    b_ref[...] = jnp.dot(h, wb_ref[...],
                         preferred_element_type=_f32).astype(b_ref.dtype)


def _finish_and_project(p0, p1, c0, c1, hprev, wr, bl, wa, wb, want_h):
    body = _finish_body if not want_h else _finish_h_body
    return pl.pallas_call(
        body,
        grid=(N // _NB,),
        in_specs=[pl.BlockSpec((_NB, D), lambda i: (i, 0)),
                  pl.BlockSpec((_NB, D), lambda i: (i, 0)),
                  pl.BlockSpec((_NB, D), lambda i: (i, 0)),
                  pl.BlockSpec((_NB, D), lambda i: (i, 0)),
                  pl.BlockSpec((_NB, D), lambda i: (i, 0)),
                  pl.BlockSpec((D, D), lambda i: (0, 0)),
                  pl.BlockSpec((1, D), lambda i: (0, 0)),
                  pl.BlockSpec((D, D), lambda i: (0, 0)),
                  pl.BlockSpec((D, D), lambda i: (0, 0))],
        out_specs=[pl.BlockSpec((_NB, D), lambda i: (i, 0)),
                   pl.BlockSpec((_NB, D), lambda i: (i, 0))],
        out_shape=[jax.ShapeDtypeStruct((N, D), _f32),
                   jax.ShapeDtypeStruct((N, D), _f32)],
    )(p0, p1, c0, c1, hprev, wr, bl, wa, wb)


def _finish_h_body(p0_ref, p1_ref, c0_ref, c1_ref, hprev_ref, wr_ref, bl_ref,
                   wa_ref, wb_ref, h_ref, b_ref):
    s = p0_ref[...] + p1_ref[...]
    cnt = c0_ref[...][:, 0:1] + c1_ref[...][:, 0:1]
    mean = s / jnp.maximum(cnt, 1.0)
    h = mean + bl_ref[...] + jnp.dot(hprev_ref[...], wr_ref[...],
                                     preferred_element_type=_f32)
    h = jnp.maximum(h, 0.0)
    h_ref[...] = h
    b_ref[...] = jnp.dot(h, wb_ref[...],
                         preferred_element_type=_f32).astype(b_ref.dtype)


def _head_body(ga_ref, gb_ref, ea_ref, we_ref, b1_ref, w2_ref, b2_ref, o_ref):
    pre = (ga_ref[...].astype(_f32) + gb_ref[...].astype(_f32) + b1_ref[...]
           + jnp.dot(ea_ref[...], we_ref[...], preferred_element_type=_f32))
    hid = jnp.maximum(pre, 0.0)
    o_ref[...] = jnp.sum(hid * w2_ref[...], axis=1, keepdims=True) + b2_ref[...]


def _head(ga, gb, ea, we, b1, w2row, b2):
    ne = ga.shape[0]
    return pl.pallas_call(
        _head_body,
        grid=(ne // _EB,),
        in_specs=[pl.BlockSpec((_EB, D), lambda i: (i, 0)),
                  pl.BlockSpec((_EB, D), lambda i: (i, 0)),
                  pl.BlockSpec((_EB, DE), lambda i: (i, 0)),
                  pl.BlockSpec((DE, D), lambda i: (0, 0)),
                  pl.BlockSpec((1, D), lambda i: (0, 0)),
                  pl.BlockSpec((1, D), lambda i: (0, 0)),
                  pl.BlockSpec((1, 1), lambda i: (0, 0))],
        out_specs=pl.BlockSpec((_EB, 1), lambda i: (i, 0)),
        out_shape=jax.ShapeDtypeStruct((ne, 1), _f32),
    )(ga, gb, ea, we, b1, w2row, b2)


def kernel(x, edge_index, edge_attr, Wl0, bl0, Wr0, Wl1, bl1, Wr1,
           W1, b1, W2, b2):
    z128 = jnp.zeros((NP, D), _f32)
    ones = jnp.ones((CH, D), _f32)
    src_ids = edge_index[0]
    dst_ids = edge_index[1]

    p0 = _mm(x, Wl0)
    P = _agg_kernel()(p0, src_ids, dst_ids, z128)
    C = _cnt_kernel()(dst_ids, ones, z128)
    h1, p1 = _finish_and_project(P[0], P[1], C[0], C[1], x, Wr0,
                                 bl0.reshape(1, D), Wl1, Wl1, want_h=True)
    P1 = _agg_kernel()(p1, src_ids, dst_ids, z128)
    A, B = _finish_and_project(P1[0], P1[1], C[0], C[1], h1, Wr1,
                               bl1.reshape(1, D), W1[:D], W1[D:2 * D],
                               want_h=False)
    GA, GB = _edge_gather_kernel()(A, B, src_ids, dst_ids)
    out = _head(GA, GB, edge_attr, W1[2 * D:], b1.reshape(1, D),
                W2.reshape(1, D), b2.reshape(1, 1))
    return out


# speedup vs baseline: 1.2027x; 1.1247x over previous
import functools

import jax
import jax.numpy as jnp
from jax import lax
from jax.experimental import pallas as pl
from jax.experimental.pallas import tpu as pltpu
from jax.experimental.pallas import tpu_sc as plsc

N = 10000
E = 320000
D = 128
DE = 16
CH = 80
NW = 32
EPW = E // NW
ITERS = EPW // CH
NP = 10240
RPW = NP // 16

_f32 = jnp.float32

@functools.lru_cache(maxsize=None)
def _sc_mesh():
    return plsc.VectorSubcoreMesh(core_axis_name="c", subcore_axis_name="s")


def _run_pipeline(iters, issue, wait_in, put_out, wait_out):
    issue(0, 0)
    issue(1, 1)
    for j in (0, 1):
        issue(j + 2, (j + 2) % 4)
        wait_in(j, j % 4)
        put_out(j, j % 4)
    for j in (2, 3):
        wait_out(j - 2, (j - 2) % 4)
        issue(j + 2, (j + 2) % 4)
        wait_in(j, j % 4)
        put_out(j, j % 4)
    lo = 4
    hi = ((iters - 3) // 4) * 4

    def body(g, carry):
        for k in range(4):
            j = lo + g * 4 + k
            wait_out(j - 2, (k + 2) % 4)
            issue(j + 2, (k + 2) % 4)
            wait_in(j, k)
            put_out(j, k)
        return carry

    lax.fori_loop(0, (hi - lo) // 4, body, 0)
    for j in range(hi, iters):
        wait_out(j - 2, (j - 2) % 4)
        if j + 2 < iters:
            issue(j + 2, (j + 2) % 4)
        wait_in(j, j % 4)
        put_out(j, j % 4)
    wait_out(iters - 2, (iters - 2) % 4)
    wait_out(iters - 1, (iters - 1) % 4)


def _agg_nocnt_body(p_hbm, src_hbm, dst_hbm, z128_hbm, agg_out,
                    sidx, didx, rows, acc_sh, semi, semd, semr, sems):
    c = lax.axis_index("c")
    s = lax.axis_index("s")
    wid = s * 2 + c
    e0 = wid * EPW
    r0 = s * RPW
    pltpu.sync_copy(z128_hbm.at[pl.ds(r0, RPW)], acc_sh.at[pl.ds(r0, RPW)])
    plsc.subcore_barrier()

    def issue(i, b):
        pltpu.async_copy(src_hbm.at[pl.ds(e0 + i * CH, CH)], sidx[b], semi[b])
        pltpu.async_copy(dst_hbm.at[pl.ds(e0 + i * CH, CH)], didx[b], semd[b])
        pltpu.make_async_copy(src_hbm.at[pl.ds(e0 + i * CH, CH)], sidx[b], semi[b]).wait()
        pltpu.async_copy(p_hbm.at[sidx[b]], rows[b], semr[b])

    def wait_in(i, b):
        pltpu.make_async_copy(dst_hbm.at[pl.ds(e0 + i * CH, CH)], didx[b], semd[b]).wait()
        pltpu.make_async_copy(p_hbm.at[sidx[b]], rows[b], semr[b]).wait()

    def put_out(i, b):
        pltpu.async_copy(rows[b], acc_sh.at[didx[b]], sems[b], add=True)

    def wait_out(i, b):
        pltpu.make_async_copy(rows[b], acc_sh.at[didx[b]], sems[b]).wait()

    _run_pipeline(ITERS, issue, wait_in, put_out, wait_out)
    plsc.subcore_barrier()
    pltpu.sync_copy(acc_sh.at[pl.ds(r0, RPW)], agg_out.at[c, pl.ds(r0, RPW)])


@functools.lru_cache(maxsize=None)
def _agg_kernel():
    return pl.kernel(
        _agg_nocnt_body,
        out_type=jax.ShapeDtypeStruct((2, NP, D), _f32),
        mesh=_sc_mesh(),
        scratch_types=[[pltpu.VMEM((CH,), jnp.int32)] * 4,
                       [pltpu.VMEM((CH,), jnp.int32)] * 4,
                       [pltpu.VMEM((CH, D), _f32)] * 4,
                       pltpu.VMEM_SHARED((NP, D), _f32),
                       [pltpu.SemaphoreType.DMA] * 4,
                       [pltpu.SemaphoreType.DMA] * 4,
                       [pltpu.SemaphoreType.DMA] * 4,
                       [pltpu.SemaphoreType.DMA] * 4])


def _cnt_body(dst_hbm, ones_hbm, z128_hbm, cnt_out,
              didx0, didx1, ones_v, acc_sh, sd0, sd1):
    c = lax.axis_index("c")
    s = lax.axis_index("s")
    wid = s * 2 + c
    e0 = wid * EPW
    r0 = s * RPW
    pltpu.sync_copy(z128_hbm.at[pl.ds(r0, RPW)], acc_sh.at[pl.ds(r0, RPW)])
    pltpu.sync_copy(ones_hbm, ones_v)
    plsc.subcore_barrier()

    def issue(i, didxb, semd):
        pltpu.async_copy(dst_hbm.at[pl.ds(e0 + i * CH, CH)], didxb, semd)

    def drain(i, didxb, semd):
        pltpu.make_async_copy(dst_hbm.at[pl.ds(e0 + i * CH, CH)], didxb, semd).wait()
        pltpu.sync_copy(ones_v, acc_sh.at[didxb], add=True)

    issue(0, didx0, sd0)

    def step(g, carry):
        i = g * 2
        issue(i + 1, didx1, sd1)
        drain(i, didx0, sd0)
        issue(i + 2, didx0, sd0)
        drain(i + 1, didx1, sd1)
        return carry

    lax.fori_loop(0, (ITERS - 1) // 2, step, 0)
    drain(ITERS - 1, didx0, sd0)
    plsc.subcore_barrier()
    pltpu.sync_copy(acc_sh.at[pl.ds(r0, RPW)], cnt_out.at[c, pl.ds(r0, RPW)])


@functools.lru_cache(maxsize=None)
def _cnt_kernel():
    return pl.kernel(
        _cnt_body,
        out_type=jax.ShapeDtypeStruct((2, NP, D), _f32),
        mesh=_sc_mesh(),
        scratch_types=[pltpu.VMEM((CH,), jnp.int32),
                       pltpu.VMEM((CH,), jnp.int32),
                       pltpu.VMEM((CH, D), _f32),
                       pltpu.VMEM_SHARED((NP, D), _f32),
                       pltpu.SemaphoreType.DMA,
                       pltpu.SemaphoreType.DMA])


def _edge_presum_body(a_hbm, b_hbm, src_hbm, dst_hbm, gs_out,
                      sidx, didx, bufa, bufb, sga, sgb, sw):
    c = lax.axis_index("c")
    s = lax.axis_index("s")
    wid = s * 2 + c
    e0 = wid * EPW
    pltpu.sync_copy(src_hbm.at[pl.ds(e0, EPW)], sidx)
    pltpu.sync_copy(dst_hbm.at[pl.ds(e0, EPW)], didx)

    def issue(i, b):
        pltpu.async_copy(a_hbm.at[sidx.at[pl.ds(i * CH, CH)]], bufa[b], sga[b])
        pltpu.async_copy(b_hbm.at[didx.at[pl.ds(i * CH, CH)]], bufb[b], sgb[b])

    def wait_in(i, b):
        pltpu.make_async_copy(a_hbm.at[sidx.at[pl.ds(i * CH, CH)]], bufa[b], sga[b]).wait()
        pltpu.make_async_copy(b_hbm.at[didx.at[pl.ds(i * CH, CH)]], bufb[b], sgb[b]).wait()

    def put_out(i, b):
        va = bufa[b]
        vb = bufb[b]

        def row(r, carry):
            for cc in range(0, D, 16):
                vb[r, pl.ds(cc, 16)] = va[r, pl.ds(cc, 16)] + vb[r, pl.ds(cc, 16)]
            return carry

        lax.fori_loop(0, CH, row, 0)
        pltpu.async_copy(vb, gs_out.at[pl.ds(e0 + i * CH, CH)], sw[b])

    def wait_out(i, b):
        pltpu.make_async_copy(bufb[b], gs_out.at[pl.ds(e0 + i * CH, CH)], sw[b]).wait()

    _run_pipeline(ITERS, issue, wait_in, put_out, wait_out)


@functools.lru_cache(maxsize=None)
def _edge_presum_kernel():
    return pl.kernel(
        _edge_presum_body,
        out_type=jax.ShapeDtypeStruct((E, D), _f32),
        mesh=_sc_mesh(),
        scratch_types=[pltpu.VMEM((EPW,), jnp.int32),
                       pltpu.VMEM((EPW,), jnp.int32),
                       [pltpu.VMEM((CH, D), _f32)] * 4,
                       [pltpu.VMEM((CH, D), _f32)] * 4,
                       [pltpu.SemaphoreType.DMA] * 4,
                       [pltpu.SemaphoreType.DMA] * 4,
                       [pltpu.SemaphoreType.DMA] * 4])


_NB = 1000
_EB = 4000


def _mm_body(x_ref, w_ref, o_ref):
    o_ref[...] = jnp.dot(x_ref[...], w_ref[...],
                         preferred_element_type=_f32)


def _mm(x, w):
    return pl.pallas_call(
        _mm_body,
        grid=(N // _NB,),
        in_specs=[pl.BlockSpec((_NB, D), lambda i: (i, 0)),
                  pl.BlockSpec((D, D), lambda i: (0, 0))],
        out_specs=pl.BlockSpec((_NB, D), lambda i: (i, 0)),
        out_shape=jax.ShapeDtypeStruct((N, D), _f32),
    )(x, w)


def _finish_body(p0_ref, p1_ref, c0_ref, c1_ref, hprev_ref, wr_ref, bl_ref,
                 wa_ref, wb_ref, a_ref, b_ref):
    s = p0_ref[...] + p1_ref[...]
    cnt = c0_ref[...][:, 0:1] + c1_ref[...][:, 0:1]
    mean = s / jnp.maximum(cnt, 1.0)
    h = mean + bl_ref[...] + jnp.dot(hprev_ref[...], wr_ref[...],
                                     preferred_element_type=_f32)
    h = jnp.maximum(h, 0.0)
    a_ref[...] = jnp.dot(h, wa_ref[...],
                         preferred_element_type=_f32).astype(a_ref.dtype)
    b_ref[...] = jnp.dot(h, wb_ref[...],
                         preferred_element_type=_f32).astype(b_ref.dtype)


def _finish_and_project(p0, p1, c0, c1, hprev, wr, bl, wa, wb, want_h):
    body = _finish_body if not want_h else _finish_h_body
    return pl.pallas_call(
        body,
        grid=(N // _NB,),
        in_specs=[pl.BlockSpec((_NB, D), lambda i: (i, 0)),
                  pl.BlockSpec((_NB, D), lambda i: (i, 0)),
                  pl.BlockSpec((_NB, D), lambda i: (i, 0)),
                  pl.BlockSpec((_NB, D), lambda i: (i, 0)),
                  pl.BlockSpec((_NB, D), lambda i: (i, 0)),
                  pl.BlockSpec((D, D), lambda i: (0, 0)),
                  pl.BlockSpec((1, D), lambda i: (0, 0)),
                  pl.BlockSpec((D, D), lambda i: (0, 0)),
                  pl.BlockSpec((D, D), lambda i: (0, 0))],
        out_specs=[pl.BlockSpec((_NB, D), lambda i: (i, 0)),
                   pl.BlockSpec((_NB, D), lambda i: (i, 0))],
        out_shape=[jax.ShapeDtypeStruct((N, D), _f32),
                   jax.ShapeDtypeStruct((N, D), _f32)],
    )(p0, p1, c0, c1, hprev, wr, bl, wa, wb)


def _finish_h_body(p0_ref, p1_ref, c0_ref, c1_ref, hprev_ref, wr_ref, bl_ref,
                   wa_ref, wb_ref, h_ref, b_ref):
    s = p0_ref[...] + p1_ref[...]
    cnt = c0_ref[...][:, 0:1] + c1_ref[...][:, 0:1]
    mean = s / jnp.maximum(cnt, 1.0)
    h = mean + bl_ref[...] + jnp.dot(hprev_ref[...], wr_ref[...],
                                     preferred_element_type=_f32)
    h = jnp.maximum(h, 0.0)
    h_ref[...] = h
    b_ref[...] = jnp.dot(h, wb_ref[...],
                         preferred_element_type=_f32).astype(b_ref.dtype)


def _head_body(gs_ref, ea_ref, we_ref, b1_ref, w2_ref, b2_ref, o_ref):
    pre = (gs_ref[...] + b1_ref[...]
           + jnp.dot(ea_ref[...], we_ref[...], preferred_element_type=_f32))
    hid = jnp.maximum(pre, 0.0)
    o_ref[...] = jnp.sum(hid * w2_ref[...], axis=1, keepdims=True) + b2_ref[...]


def _head(gs, ea, we, b1, w2row, b2):
    ne = gs.shape[0]
    return pl.pallas_call(
        _head_body,
        grid=(ne // _EB,),
        in_specs=[pl.BlockSpec((_EB, D), lambda i: (i, 0)),
                  pl.BlockSpec((_EB, DE), lambda i: (i, 0)),
                  pl.BlockSpec((DE, D), lambda i: (0, 0)),
                  pl.BlockSpec((1, D), lambda i: (0, 0)),
                  pl.BlockSpec((1, D), lambda i: (0, 0)),
                  pl.BlockSpec((1, 1), lambda i: (0, 0))],
        out_specs=pl.BlockSpec((_EB, 1), lambda i: (i, 0)),
        out_shape=jax.ShapeDtypeStruct((ne, 1), _f32),
    )(gs, ea, we, b1, w2row, b2)


def kernel(x, edge_index, edge_attr, Wl0, bl0, Wr0, Wl1, bl1, Wr1,
           W1, b1, W2, b2):
    z128 = jnp.zeros((NP, D), _f32)
    ones = jnp.ones((CH, D), _f32)
    src_ids = edge_index[0]
    dst_ids = edge_index[1]

    p0 = _mm(x, Wl0)
    P = _agg_kernel()(p0, src_ids, dst_ids, z128)
    C = _cnt_kernel()(dst_ids, ones, z128)
    h1, p1 = _finish_and_project(P[0], P[1], C[0], C[1], x, Wr0,
                                 bl0.reshape(1, D), Wl1, Wl1, want_h=True)
    P1 = _agg_kernel()(p1, src_ids, dst_ids, z128)
    A, B = _finish_and_project(P1[0], P1[1], C[0], C[1], h1, Wr1,
                               bl1.reshape(1, D), W1[:D], W1[D:2 * D],
                               want_h=False)
    GS = _edge_presum_kernel()(A, B, src_ids, dst_ids)
    out = _head(GS, edge_attr, W1[2 * D:], b1.reshape(1, D),
                W2.reshape(1, D), b2.reshape(1, 1))
    return out
